# Initial kernel scaffold; baseline (speedup 1.0000x reference)
import jax
import jax.numpy as jnp
from jax.experimental import pallas as pl


def kernel(embeddings, laplacians, boundaries, conv1_W0, conv1_W1, conv1_W2, conv1_W3, attn_Wq, attn_Wk, attn_Wv, conv2_W0, conv2_W1, conv2_W2, conv2_W3, lin_W, lin_b, order, idx):
    raise NotImplementedError("write your pallas kernel here")


# speedup vs baseline: 4.1006x; 4.1006x over previous
import jax
import jax.numpy as jnp
from jax.experimental import pallas as pl
from jax.experimental.pallas import tpu as pltpu

N = 4096
D = 20
BR = 512
NB = N // BR


def _conv1_kernel(s_ref, lap_ref, bd_ref, bu_ref, x_ref, xd_ref, xu_ref,
                  w0_ref, w1_ref, w2_ref, w3_ref, e1_ref):
    i = pl.program_id(0)
    x_blk = x_ref[pl.ds(i * BR, BR), :]
    acc = jnp.dot(x_blk, w0_ref[...], preferred_element_type=jnp.float32)
    acc += jnp.dot(jnp.dot(lap_ref[0], x_ref[...], preferred_element_type=jnp.float32),
                   w1_ref[...], preferred_element_type=jnp.float32)
    acc += jnp.dot(jnp.dot(bd_ref[0], xd_ref[...], preferred_element_type=jnp.float32),
                   w2_ref[...], preferred_element_type=jnp.float32)
    acc += jnp.dot(jnp.dot(bu_ref[0], xu_ref[...], preferred_element_type=jnp.float32),
                   w3_ref[...], preferred_element_type=jnp.float32)
    e1_ref[...] = jnp.tanh(acc)


def _attn_kernel(s_ref, lap_ref, e1_ref, wq_ref, wk_ref, wv_ref, e2_ref):
    i = pl.program_id(0)
    e1 = e1_ref[...]
    q = jnp.dot(e1_ref[pl.ds(i * BR, BR), :], wq_ref[...], preferred_element_type=jnp.float32)
    k = jnp.dot(e1, wk_ref[...], preferred_element_type=jnp.float32)
    v = jnp.dot(e1, wv_ref[...], preferred_element_type=jnp.float32)
    scores = jax.lax.dot_general(q, k, (((1,), (1,)), ((), ())),
                                 preferred_element_type=jnp.float32)
    scores = scores * (1.0 / (D ** 0.5))
    scores = jnp.where(lap_ref[0] != 0.0, scores, -1e9)
    m = jnp.max(scores, axis=1, keepdims=True)
    e = jnp.exp(scores - m)
    a = e / jnp.sum(e, axis=1, keepdims=True)
    e2_ref[...] = jnp.dot(a, v, preferred_element_type=jnp.float32)


def _conv2_kernel(lrow_ref, bdrow_ref, burow_ref, e2row_ref, e2_ref,
                  xd_ref, xu_ref, w0_ref, w1_ref, w2_ref, w3_ref,
                  lw_ref, lb_ref, out_ref):
    acc = jnp.dot(e2row_ref[...], w0_ref[...], preferred_element_type=jnp.float32)
    acc += jnp.dot(jnp.dot(lrow_ref[...], e2_ref[...], preferred_element_type=jnp.float32),
                   w1_ref[...], preferred_element_type=jnp.float32)
    acc += jnp.dot(jnp.dot(bdrow_ref[...], xd_ref[...], preferred_element_type=jnp.float32),
                   w2_ref[...], preferred_element_type=jnp.float32)
    acc += jnp.dot(jnp.dot(burow_ref[...], xu_ref[...], preferred_element_type=jnp.float32),
                   w3_ref[...], preferred_element_type=jnp.float32)
    out_ref[...] = jnp.dot(jnp.tanh(acc), lw_ref[...],
                           preferred_element_type=jnp.float32) + lb_ref[...]


def kernel(embeddings, laplacians, boundaries,
           conv1_W0, conv1_W1, conv1_W2, conv1_W3,
           attn_Wq, attn_Wk, attn_Wv,
           conv2_W0, conv2_W1, conv2_W2, conv2_W3,
           lin_W, lin_b, order, idx):
    f32 = jnp.float32
    order = jnp.asarray(order, jnp.int32)
    idx = jnp.asarray(idx, jnp.int32)
    od = jnp.clip(order - 1, 0, 3)
    ou = jnp.clip(order + 1, 0, 3)
    have_d = (order > 0).astype(f32)
    have_u = (order < 3).astype(f32)

    x = jax.lax.dynamic_index_in_dim(embeddings, order, 0, keepdims=False)
    xd = jax.lax.dynamic_index_in_dim(embeddings, od, 0, keepdims=False)
    xu = jax.lax.dynamic_index_in_dim(embeddings, ou, 0, keepdims=False)
    w2a = conv1_W2 * have_d
    w3a = conv1_W3 * have_u
    w2b = conv2_W2 * have_d
    w3b = conv2_W3 * have_u

    scalars = jnp.stack([order, od, ou, idx])

    def full2(shape):
        return pl.BlockSpec(shape, lambda i, s: (0, 0))

    def big(which):
        return pl.BlockSpec((1, BR, N), lambda i, s: (s[which], i, 0))

    e1 = pl.pallas_call(
        _conv1_kernel,
        grid_spec=pltpu.PrefetchScalarGridSpec(
            num_scalar_prefetch=1,
            grid=(NB,),
            in_specs=[
                big(0),
                big(1),
                big(2),
                full2((N, D)),
                full2((N, D)),
                full2((N, D)),
                full2((D, D)), full2((D, D)), full2((D, D)), full2((D, D)),
            ],
            out_specs=pl.BlockSpec((BR, D), lambda i, s: (i, 0)),
        ),
        out_shape=jax.ShapeDtypeStruct((N, D), f32),
    )(scalars, laplacians, boundaries, boundaries, x, xd, xu,
      conv1_W0, conv1_W1, w2a, w3a)

    e2 = pl.pallas_call(
        _attn_kernel,
        grid_spec=pltpu.PrefetchScalarGridSpec(
            num_scalar_prefetch=1,
            grid=(NB,),
            in_specs=[
                big(0),
                full2((N, D)),
                full2((D, D)), full2((D, D)), full2((D, D)),
            ],
            out_specs=pl.BlockSpec((BR, D), lambda i, s: (i, 0)),
        ),
        out_shape=jax.ShapeDtypeStruct((N, D), f32),
    )(scalars, laplacians, e1, attn_Wq, attn_Wk, attn_Wv)

    lrow = jax.lax.dynamic_slice(laplacians, (order, idx, 0), (1, 1, N)).reshape(1, N)
    bdrow = jax.lax.dynamic_slice(boundaries, (od, idx, 0), (1, 1, N)).reshape(1, N)
    burow = jax.lax.dynamic_slice(boundaries, (ou, idx, 0), (1, 1, N)).reshape(1, N)
    e2row = jax.lax.dynamic_slice(e2, (idx, 0), (1, D))

    out = pl.pallas_call(
        _conv2_kernel,
        out_shape=jax.ShapeDtypeStruct((1, 10), f32),
    )(lrow, bdrow, burow, e2row, e2, xd, xu,
      conv2_W0, conv2_W1, w2b, w3b, lin_W, lin_b.reshape(1, 10))

    return out.reshape(10)


# speedup vs baseline: 4.9350x; 1.2035x over previous
import jax
import jax.numpy as jnp
from jax.experimental import pallas as pl
from jax.experimental.pallas import tpu as pltpu

N = 4096
D = 20
BR = 512
NB = N // BR


def _conv1_kernel(s_ref, lap_ref, bd_ref, bu_ref, x_ref, xd_ref, xu_ref,
                  w0_ref, w1_ref, w2_ref, w3_ref, e1_ref):
    i = pl.program_id(0)
    x_blk = x_ref[pl.ds(i * BR, BR), :]
    acc = jnp.dot(x_blk, w0_ref[...], preferred_element_type=jnp.float32)
    acc += jnp.dot(jnp.dot(lap_ref[0], x_ref[...], preferred_element_type=jnp.float32),
                   w1_ref[...], preferred_element_type=jnp.float32)
    acc += jnp.dot(jnp.dot(bd_ref[0], xd_ref[...], preferred_element_type=jnp.float32),
                   w2_ref[...], preferred_element_type=jnp.float32)
    acc += jnp.dot(jnp.dot(bu_ref[0], xu_ref[...], preferred_element_type=jnp.float32),
                   w3_ref[...], preferred_element_type=jnp.float32)
    e1_ref[...] = jnp.tanh(acc)


def _attn_kernel(s_ref, lap_ref, e1_ref, wq_ref, wk_ref, wv_ref, e2_ref):
    i = pl.program_id(0)
    e1 = e1_ref[...]
    q = jnp.dot(e1_ref[pl.ds(i * BR, BR), :], wq_ref[...], preferred_element_type=jnp.float32)
    k = jnp.dot(e1, wk_ref[...], preferred_element_type=jnp.float32)
    v = jnp.dot(e1, wv_ref[...], preferred_element_type=jnp.float32)
    v1 = jnp.concatenate([v, jnp.ones((N, 1), jnp.float32)], axis=1)
    scores = jax.lax.dot_general(q, k, (((1,), (1,)), ((), ())),
                                 preferred_element_type=jnp.float32)
    e = jnp.exp(scores)
    w = jnp.where(lap_ref[0] != 0.0, e, 0.0)
    wv = jnp.dot(w, v1, preferred_element_type=jnp.float32)
    e2_ref[...] = wv[:, :D] / wv[:, D:D + 1]


def _conv2_kernel(lrow_ref, bdrow_ref, burow_ref, e2row_ref, e2_ref,
                  xd_ref, xu_ref, w0_ref, w1_ref, w2_ref, w3_ref,
                  lw_ref, lb_ref, out_ref):
    acc = jnp.dot(e2row_ref[...], w0_ref[...], preferred_element_type=jnp.float32)
    acc += jnp.dot(jnp.dot(lrow_ref[...], e2_ref[...], preferred_element_type=jnp.float32),
                   w1_ref[...], preferred_element_type=jnp.float32)
    acc += jnp.dot(jnp.dot(bdrow_ref[...], xd_ref[...], preferred_element_type=jnp.float32),
                   w2_ref[...], preferred_element_type=jnp.float32)
    acc += jnp.dot(jnp.dot(burow_ref[...], xu_ref[...], preferred_element_type=jnp.float32),
                   w3_ref[...], preferred_element_type=jnp.float32)
    out_ref[...] = jnp.dot(jnp.tanh(acc), lw_ref[...],
                           preferred_element_type=jnp.float32) + lb_ref[...]


def kernel(embeddings, laplacians, boundaries,
           conv1_W0, conv1_W1, conv1_W2, conv1_W3,
           attn_Wq, attn_Wk, attn_Wv,
           conv2_W0, conv2_W1, conv2_W2, conv2_W3,
           lin_W, lin_b, order, idx):
    f32 = jnp.float32
    order = jnp.asarray(order, jnp.int32)
    idx = jnp.asarray(idx, jnp.int32)
    od = jnp.clip(order - 1, 0, 3)
    ou = jnp.clip(order + 1, 0, 3)
    have_d = (order > 0).astype(f32)
    have_u = (order < 3).astype(f32)

    x = jax.lax.dynamic_index_in_dim(embeddings, order, 0, keepdims=False)
    xd = jax.lax.dynamic_index_in_dim(embeddings, od, 0, keepdims=False)
    xu = jax.lax.dynamic_index_in_dim(embeddings, ou, 0, keepdims=False)
    w2a = conv1_W2 * have_d
    w3a = conv1_W3 * have_u
    w2b = conv2_W2 * have_d
    w3b = conv2_W3 * have_u

    scalars = jnp.stack([order, od, ou, idx])

    def full2(shape):
        return pl.BlockSpec(shape, lambda i, s: (0, 0))

    def big(which):
        return pl.BlockSpec((1, BR, N), lambda i, s: (s[which], i, 0))

    e1 = pl.pallas_call(
        _conv1_kernel,
        grid_spec=pltpu.PrefetchScalarGridSpec(
            num_scalar_prefetch=1,
            grid=(NB,),
            in_specs=[
                big(0),
                big(1),
                big(2),
                full2((N, D)),
                full2((N, D)),
                full2((N, D)),
                full2((D, D)), full2((D, D)), full2((D, D)), full2((D, D)),
            ],
            out_specs=pl.BlockSpec((BR, D), lambda i, s: (i, 0)),
        ),
        out_shape=jax.ShapeDtypeStruct((N, D), f32),
    )(scalars, laplacians, boundaries, boundaries, x, xd, xu,
      conv1_W0, conv1_W1, w2a, w3a)

    e2 = pl.pallas_call(
        _attn_kernel,
        grid_spec=pltpu.PrefetchScalarGridSpec(
            num_scalar_prefetch=1,
            grid=(NB,),
            in_specs=[
                big(0),
                full2((N, D)),
                full2((D, D)), full2((D, D)), full2((D, D)),
            ],
            out_specs=pl.BlockSpec((BR, D), lambda i, s: (i, 0)),
        ),
        out_shape=jax.ShapeDtypeStruct((N, D), f32),
    )(scalars, laplacians, e1, attn_Wq * (1.0 / (D ** 0.5)), attn_Wk, attn_Wv)

    lrow = jax.lax.dynamic_slice(laplacians, (order, idx, 0), (1, 1, N)).reshape(1, N)
    bdrow = jax.lax.dynamic_slice(boundaries, (od, idx, 0), (1, 1, N)).reshape(1, N)
    burow = jax.lax.dynamic_slice(boundaries, (ou, idx, 0), (1, 1, N)).reshape(1, N)
    e2row = jax.lax.dynamic_slice(e2, (idx, 0), (1, D))

    out = pl.pallas_call(
        _conv2_kernel,
        out_shape=jax.ShapeDtypeStruct((1, 10), f32),
    )(lrow, bdrow, burow, e2row, e2, xd, xu,
      conv2_W0, conv2_W1, w2b, w3b, lin_W, lin_b.reshape(1, 10))

    return out.reshape(10)


# speedup vs baseline: 5.1057x; 1.0346x over previous
import jax
import jax.numpy as jnp
from jax.experimental import pallas as pl
from jax.experimental.pallas import tpu as pltpu

N = 4096
D = 20
BR = 512
NB = N // BR


def _conv1_kernel(s_ref, lap_ref, bd_ref, bu_ref, x_ref, xd_ref, xu_ref,
                  w0_ref, w1_ref, w2_ref, w3_ref, wq_ref, wk_ref, wv_ref,
                  q_ref, k_ref, v1_ref):
    i = pl.program_id(0)
    x_blk = x_ref[pl.ds(i * BR, BR), :]
    acc = jnp.dot(x_blk, w0_ref[...], preferred_element_type=jnp.float32)
    acc += jnp.dot(jnp.dot(lap_ref[0], x_ref[...], preferred_element_type=jnp.float32),
                   w1_ref[...], preferred_element_type=jnp.float32)
    acc += jnp.dot(jnp.dot(bd_ref[0], xd_ref[...], preferred_element_type=jnp.float32),
                   w2_ref[...], preferred_element_type=jnp.float32)
    acc += jnp.dot(jnp.dot(bu_ref[0], xu_ref[...], preferred_element_type=jnp.float32),
                   w3_ref[...], preferred_element_type=jnp.float32)
    e1 = jnp.tanh(acc)
    q_ref[...] = jnp.dot(e1, wq_ref[...], preferred_element_type=jnp.float32)
    k_ref[...] = jnp.dot(e1, wk_ref[...], preferred_element_type=jnp.float32)
    v = jnp.dot(e1, wv_ref[...], preferred_element_type=jnp.float32)
    v1_ref[...] = jnp.concatenate([v, jnp.ones((BR, 1), jnp.float32)], axis=1)


def _attn_kernel(s_ref, lap_ref, q_ref, k_ref, v1_ref, e2_ref):
    scores = jax.lax.dot_general(q_ref[...], k_ref[...], (((1,), (1,)), ((), ())),
                                 preferred_element_type=jnp.float32)
    e = jnp.exp(scores)
    w = jnp.where(lap_ref[0] != 0.0, e, 0.0)
    wv = jnp.dot(w, v1_ref[...], preferred_element_type=jnp.float32)
    e2_ref[...] = wv[:, :D] / wv[:, D:D + 1]


def _conv2_kernel(lrow_ref, bdrow_ref, burow_ref, e2row_ref, e2_ref,
                  xd_ref, xu_ref, w0_ref, w1_ref, w2_ref, w3_ref,
                  lw_ref, lb_ref, out_ref):
    acc = jnp.dot(e2row_ref[...], w0_ref[...], preferred_element_type=jnp.float32)
    acc += jnp.dot(jnp.dot(lrow_ref[...], e2_ref[...], preferred_element_type=jnp.float32),
                   w1_ref[...], preferred_element_type=jnp.float32)
    acc += jnp.dot(jnp.dot(bdrow_ref[...], xd_ref[...], preferred_element_type=jnp.float32),
                   w2_ref[...], preferred_element_type=jnp.float32)
    acc += jnp.dot(jnp.dot(burow_ref[...], xu_ref[...], preferred_element_type=jnp.float32),
                   w3_ref[...], preferred_element_type=jnp.float32)
    out_ref[...] = jnp.dot(jnp.tanh(acc), lw_ref[...],
                           preferred_element_type=jnp.float32) + lb_ref[...]


def kernel(embeddings, laplacians, boundaries,
           conv1_W0, conv1_W1, conv1_W2, conv1_W3,
           attn_Wq, attn_Wk, attn_Wv,
           conv2_W0, conv2_W1, conv2_W2, conv2_W3,
           lin_W, lin_b, order, idx):
    f32 = jnp.float32
    order = jnp.asarray(order, jnp.int32)
    idx = jnp.asarray(idx, jnp.int32)
    od = jnp.clip(order - 1, 0, 3)
    ou = jnp.clip(order + 1, 0, 3)
    have_d = (order > 0).astype(f32)
    have_u = (order < 3).astype(f32)

    x = jax.lax.dynamic_index_in_dim(embeddings, order, 0, keepdims=False)
    xd = jax.lax.dynamic_index_in_dim(embeddings, od, 0, keepdims=False)
    xu = jax.lax.dynamic_index_in_dim(embeddings, ou, 0, keepdims=False)
    w2a = conv1_W2 * have_d
    w3a = conv1_W3 * have_u
    w2b = conv2_W2 * have_d
    w3b = conv2_W3 * have_u

    scalars = jnp.stack([order, od, ou, idx])

    def full2(shape):
        return pl.BlockSpec(shape, lambda i, s: (0, 0))

    def big(which):
        return pl.BlockSpec((1, BR, N), lambda i, s: (s[which], i, 0))

    q, k, v1 = pl.pallas_call(
        _conv1_kernel,
        grid_spec=pltpu.PrefetchScalarGridSpec(
            num_scalar_prefetch=1,
            grid=(NB,),
            in_specs=[
                big(0),
                big(1),
                big(2),
                full2((N, D)),
                full2((N, D)),
                full2((N, D)),
                full2((D, D)), full2((D, D)), full2((D, D)), full2((D, D)),
                full2((D, D)), full2((D, D)), full2((D, D)),
            ],
            out_specs=[
                pl.BlockSpec((BR, D), lambda i, s: (i, 0)),
                pl.BlockSpec((BR, D), lambda i, s: (i, 0)),
                pl.BlockSpec((BR, D + 1), lambda i, s: (i, 0)),
            ],
        ),
        out_shape=[
            jax.ShapeDtypeStruct((N, D), f32),
            jax.ShapeDtypeStruct((N, D), f32),
            jax.ShapeDtypeStruct((N, D + 1), f32),
        ],
    )(scalars, laplacians, boundaries, boundaries, x, xd, xu,
      conv1_W0, conv1_W1, w2a, w3a,
      attn_Wq * (1.0 / (D ** 0.5)), attn_Wk, attn_Wv)

    e2 = pl.pallas_call(
        _attn_kernel,
        grid_spec=pltpu.PrefetchScalarGridSpec(
            num_scalar_prefetch=1,
            grid=(NB,),
            in_specs=[
                big(0),
                pl.BlockSpec((BR, D), lambda i, s: (i, 0)),
                full2((N, D)),
                full2((N, D + 1)),
            ],
            out_specs=pl.BlockSpec((BR, D), lambda i, s: (i, 0)),
        ),
        out_shape=jax.ShapeDtypeStruct((N, D), f32),
    )(scalars, laplacians, q, k, v1)

    lrow = jax.lax.dynamic_slice(laplacians, (order, idx, 0), (1, 1, N)).reshape(1, N)
    bdrow = jax.lax.dynamic_slice(boundaries, (od, idx, 0), (1, 1, N)).reshape(1, N)
    burow = jax.lax.dynamic_slice(boundaries, (ou, idx, 0), (1, 1, N)).reshape(1, N)
    e2row = jax.lax.dynamic_slice(e2, (idx, 0), (1, D))

    out = pl.pallas_call(
        _conv2_kernel,
        out_shape=jax.ShapeDtypeStruct((1, 10), f32),
    )(lrow, bdrow, burow, e2row, e2, xd, xu,
      conv2_W0, conv2_W1, w2b, w3b, lin_W, lin_b.reshape(1, 10))

    return out.reshape(10)
